# baseline (speedup 1.0000x reference)
import functools

import jax
import jax.numpy as jnp
from jax import lax
from jax.experimental import pallas as pl
from jax.experimental.pallas import tpu as pltpu
from jax.experimental.pallas import tpu_sc as plsc

_N = 8192
_BS, _C, _D, _H, _W = 2, 2, 32, 512, 512
_SZ, _SY, _SX = 7, 15, 15
_SCALE = 10000.0

_NC = 2
_NS = 16
_L = 16

_ROUND_Z = 1
_NROUNDS = _D // _ROUND_Z
_SLAB_WORDS = _C * _ROUND_Z * _H * _W
_TEC_WB = _SLAB_WORDS // _NS
_DHW = _D * _H * _W
_HW = _H * _W
_CHUNK = _N // _NS
_NBUF_ROWS = 60
_DUMP = _SLAB_WORDS
_ZERO_W = 16384
_ACC_EXTRA = 7424


def _body(lb, lc, lz, ly, lx, xo, yo, zo, it, pad, cf,
          out,
          acc_a, acc_b,
          pad_v, lb_v, lc_v, lz_v, ly_v, lx_v,
          xo_v, yo_v, zo_v, it_v, cf_v,
          zero_v, val_a, idx_a, val_b, idx_b, sem, zsem, wbsem):
    cid = lax.axis_index("c")
    sid = lax.axis_index("s")
    base_e = sid * _CHUNK

    def sload(ref, i):
        return ref[pl.ds(i, _L)][0]

    pltpu.sync_copy(pad, pad_v)
    pltpu.sync_copy(cf, cf_v)
    pltpu.sync_copy(lb.at[pl.ds(base_e, _CHUNK)], lb_v.at[pl.ds(0, _CHUNK)])
    pltpu.sync_copy(lc.at[pl.ds(base_e, _CHUNK)], lc_v.at[pl.ds(0, _CHUNK)])
    pltpu.sync_copy(lz.at[pl.ds(base_e, _CHUNK)], lz_v.at[pl.ds(0, _CHUNK)])
    pltpu.sync_copy(ly.at[pl.ds(base_e, _CHUNK)], ly_v.at[pl.ds(0, _CHUNK)])
    pltpu.sync_copy(lx.at[pl.ds(base_e, _CHUNK)], lx_v.at[pl.ds(0, _CHUNK)])
    pltpu.sync_copy(xo.at[pl.ds(base_e, _CHUNK)], xo_v.at[pl.ds(0, _CHUNK)])
    pltpu.sync_copy(yo.at[pl.ds(base_e, _CHUNK)], yo_v.at[pl.ds(0, _CHUNK)])
    pltpu.sync_copy(zo.at[pl.ds(base_e, _CHUNK)], zo_v.at[pl.ds(0, _CHUNK)])
    pltpu.sync_copy(it.at[pl.ds(base_e, _CHUNK)], it_v.at[pl.ds(0, _CHUNK)])

    ii = lax.iota(jnp.int32, _L)
    zvec = jnp.zeros((_L,), jnp.float32)

    def zb(j, _):
        zero_v[pl.ds(j * _L, _L)] = zvec
        return 0
    lax.fori_loop(0, _ZERO_W // _L, zb, 0)

    dump_idx = _DUMP + ii

    def fire_zero(acc_ref, zs):
        def zr(j, _):
            pltpu.async_copy(
                zero_v,
                acc_ref.at[pl.ds(sid * _TEC_WB + j * _ZERO_W, _ZERO_W)], zs)
            return 0
        lax.fori_loop(0, _TEC_WB // _ZERO_W, zr, 0)

    def wait_zero(acc_ref, zs):
        def zw(j, _):
            pltpu.make_async_copy(
                zero_v,
                acc_ref.at[pl.ds(sid * _TEC_WB + j * _ZERO_W, _ZERO_W)],
                zs).wait()
            return 0
        lax.fori_loop(0, _TEC_WB // _ZERO_W, zw, 0)

    bcl = sid // 8
    yo8 = sid % 8

    def wb_refs(acc_ref, z0):
        hbm_off = ((2 * cid + bcl) * _DHW + z0 * _HW + yo8 * (_H // 8) * _W)
        return (acc_ref.at[pl.ds(sid * _TEC_WB, _TEC_WB)],
                out.at[pl.ds(hbm_off, _TEC_WB)])

    fire_zero(acc_a, zsem.at[0])

    def phase(z0, acc, zs, wbs, scatter_body, maint):
        wait_zero(acc, zs)
        plsc.subcore_barrier()
        scatter_body(z0, acc)
        maint()
        plsc.subcore_barrier()
        src_r, dst_r = wb_refs(acc, z0)
        pltpu.async_copy(src_r, dst_r, wbs)

    def scatter_body(z0, acc):
        def emitter_body(i, carry):
            elz = sload(lz_v, i)
            elb = sload(lb_v, i)
            zlo = jnp.maximum(z0, elz - (_SZ // 2))
            zhi = jnp.minimum(z0 + _ROUND_Z - 1, elz + (_SZ // 2))
            zub = jnp.where(elb == cid, zhi + 1, zlo)

            def z_body(zz, carry):
                rowcnt, fcnt = carry
                elc = sload(lc_v, i)
                ely = sload(ly_v, i)
                elx = sload(lx_v, i)
                dz = sload(zo_v, i) - 0.5
                dy = sload(yo_v, i) - 0.5
                dx = sload(xo_v, i) - 0.5
                fzi = jnp.where(dz < 0.0, -1, 0)
                fyi = jnp.where(dy < 0.0, -1, 0)
                fxi = jnp.where(dx < 0.0, -1, 0)
                wz1 = dz - fzi.astype(jnp.float32)
                wy1 = dy - fyi.astype(jnp.float32)
                wx1 = dx - fxi.astype(jnp.float32)
                wz0 = 1.0 - wz1
                wy0 = 1.0 - wy1
                wx0 = 1.0 - wx1
                cfv = cf_v[...]
                fac = sload(it_v, i) * _SCALE * jnp.where(elc == 0, cfv[0],
                                                          cfv[1])
                wzf0 = wz0 * fac
                wzf1 = wz1 * fac

                k = zz - elz + (_SZ // 2)
                jz0 = k + 1 + fzi
                jz1 = jz0 + 1
                x0 = 1 + fxi
                x1 = x0 + 1
                sbase_z = (elc * _ROUND_Z + (zz - z0)) * _HW
                xbase = elx - (_SX // 2)
                xv = xbase + ii
                xok = (xv >= 0) & (xv < _W) & (ii < _SX)

                dylo = jnp.maximum(0, (_SY // 2) - ely)
                dyhi = jnp.minimum(_SY - 1, (_H - 1) - ely + (_SY // 2))

                bv = []
                for t in range(_SY + 1):
                    jy = t + 1 + fyi
                    xb0 = (pad_v[jz0, jy, pl.ds(x0, _L)] * wx0 +
                           pad_v[jz0, jy, pl.ds(x1, _L)] * wx1)
                    xb1 = (pad_v[jz1, jy, pl.ds(x0, _L)] * wx0 +
                           pad_v[jz1, jy, pl.ds(x1, _L)] * wx1)
                    bv.append(xb0 * wzf0 + xb1 * wzf1)

                y0 = ely - (_SY // 2)
                idx0 = jnp.where(xok, sbase_z + y0 * _W + xv, dump_idx)
                p = fcnt & 1
                base_pos = rowcnt * _L
                rows = []
                for t in range(_SY):
                    row = bv[t] * wy0 + bv[t + 1] * wy1
                    yok = (t >= dylo) & (t <= dyhi)
                    ridx = jnp.where(yok, idx0 + t * _W, dump_idx)
                    rows.append((row, ridx))

                def stage(val_r, idx_r):
                    for t, (row, ridx) in enumerate(rows):
                        val_r[pl.ds(base_pos + t * _L, _L)] = row
                        idx_r[pl.ds(base_pos + t * _L, _L)] = ridx

                @pl.when(p == 0)
                def _():
                    stage(val_a, idx_a)

                @pl.when(p == 1)
                def _():
                    stage(val_b, idx_b)

                rowcnt = rowcnt + _SY
                full = rowcnt == _NBUF_ROWS

                @pl.when(full & (p == 0))
                def _():
                    pltpu.async_copy(val_a, acc.at[idx_a], sem.at[0],
                                     add=True)

                    @pl.when(fcnt >= 1)
                    def _():
                        pltpu.make_async_copy(val_b, acc.at[idx_b],
                                              sem.at[1]).wait()

                @pl.when(full & (p == 1))
                def _():
                    pltpu.async_copy(val_b, acc.at[idx_b], sem.at[1],
                                     add=True)
                    pltpu.make_async_copy(val_a, acc.at[idx_a],
                                          sem.at[0]).wait()

                return (jnp.where(full, 0, rowcnt),
                        jnp.where(full, fcnt + 1, fcnt))

            return lax.fori_loop(zlo, zub, z_body, carry)

        rowcnt, fcnt = lax.fori_loop(0, _CHUNK, emitter_body, (0, 0))

        pf = fcnt & 1

        @pl.when(pf == 0)
        def _():
            def pad_body(j, _):
                pos = j * _L
                val_a[pl.ds(pos, _L)] = zvec
                idx_a[pl.ds(pos, _L)] = dump_idx
                return 0
            lax.fori_loop(rowcnt, _NBUF_ROWS, pad_body, 0)
            pltpu.sync_copy(val_a, acc.at[idx_a], add=True)

            @pl.when(fcnt >= 1)
            def _():
                pltpu.make_async_copy(val_b, acc.at[idx_b], sem.at[1]).wait()

        @pl.when(pf == 1)
        def _():
            def pad_body(j, _):
                pos = j * _L
                val_b[pl.ds(pos, _L)] = zvec
                idx_b[pl.ds(pos, _L)] = dump_idx
                return 0
            lax.fori_loop(rowcnt, _NBUF_ROWS, pad_body, 0)
            pltpu.sync_copy(val_b, acc.at[idx_b], add=True)
            pltpu.make_async_copy(val_a, acc.at[idx_a], sem.at[0]).wait()

    def pair_body(kk, _):
        def maint_a():
            @pl.when(kk >= 1)
            def _():
                src_r, dst_r = wb_refs(acc_b, 2 * kk - 1)
                pltpu.make_async_copy(src_r, dst_r, wbsem.at[1]).wait()
            fire_zero(acc_b, zsem.at[1])

        phase(2 * kk, acc_a, zsem.at[0], wbsem.at[0], scatter_body, maint_a)

        def maint_b():
            src_r, dst_r = wb_refs(acc_a, 2 * kk)
            pltpu.make_async_copy(src_r, dst_r, wbsem.at[0]).wait()

            @pl.when(kk <= (_NROUNDS // 2) - 2)
            def _():
                fire_zero(acc_a, zsem.at[0])

        phase(2 * kk + 1, acc_b, zsem.at[1], wbsem.at[1], scatter_body,
              maint_b)
        return 0

    lax.fori_loop(0, _NROUNDS // 2, pair_body, 0)

    src_r, dst_r = wb_refs(acc_b, _NROUNDS - 1)
    pltpu.make_async_copy(src_r, dst_r, wbsem.at[1]).wait()


@jax.jit
def _sc_place(lb, lc, lz, ly, lx, xo, yo, zo, it, pad, cf):
    mesh = plsc.VectorSubcoreMesh(core_axis_name="c", subcore_axis_name="s",
                                  num_cores=_NC, num_subcores=_NS)
    f = pl.kernel(
        _body,
        out_type=jax.ShapeDtypeStruct((_BS * _C * _D * _H * _W,), jnp.float32),
        mesh=mesh,
        scratch_types=[
            pltpu.VMEM_SHARED((_SLAB_WORDS + _ACC_EXTRA,), jnp.float32),
            pltpu.VMEM_SHARED((_SLAB_WORDS + _ACC_EXTRA,), jnp.float32),
            pltpu.VMEM((_SZ + 2, _SY + 2, 24), jnp.float32),
            pltpu.VMEM((_CHUNK + _L,), jnp.int32),
            pltpu.VMEM((_CHUNK + _L,), jnp.int32),
            pltpu.VMEM((_CHUNK + _L,), jnp.int32),
            pltpu.VMEM((_CHUNK + _L,), jnp.int32),
            pltpu.VMEM((_CHUNK + _L,), jnp.int32),
            pltpu.VMEM((_CHUNK + _L,), jnp.float32),
            pltpu.VMEM((_CHUNK + _L,), jnp.float32),
            pltpu.VMEM((_CHUNK + _L,), jnp.float32),
            pltpu.VMEM((_CHUNK + _L,), jnp.float32),
            pltpu.VMEM((_L,), jnp.float32),
            pltpu.VMEM((_ZERO_W,), jnp.float32),
            pltpu.VMEM((_NBUF_ROWS * _L,), jnp.float32),
            pltpu.VMEM((_NBUF_ROWS * _L,), jnp.int32),
            pltpu.VMEM((_NBUF_ROWS * _L,), jnp.float32),
            pltpu.VMEM((_NBUF_ROWS * _L,), jnp.int32),
            pltpu.SemaphoreType.DMA((2,)),
            pltpu.SemaphoreType.DMA((2,)),
            pltpu.SemaphoreType.DMA((2,)),
        ],
    )
    return f(lb, lc, lz, ly, lx, xo, yo, zo, it, pad, cf)


def kernel(loc_b, loc_c, loc_z, loc_y, loc_x, x_os, y_os, z_os, ints,
           psf_volume, channel_facs):
    psfc = jnp.maximum(psf_volume.astype(jnp.float32), 0.0)
    pad = jnp.pad(psfc, ((1, 1), (1, 1), (1, 24 - _SX - 1)))
    cf = jnp.zeros((_L,), jnp.float32).at[:_C].set(
        channel_facs.astype(jnp.float32))
    out = _sc_place(
        loc_b.astype(jnp.int32), loc_c.astype(jnp.int32),
        loc_z.astype(jnp.int32), loc_y.astype(jnp.int32),
        loc_x.astype(jnp.int32),
        x_os.astype(jnp.float32), y_os.astype(jnp.float32),
        z_os.astype(jnp.float32), ints.astype(jnp.float32),
        pad, cf)
    return out.reshape(_BS, _C, _D, _H, _W)


# speedup vs baseline: 1.1834x; 1.1834x over previous
import functools

import jax
import jax.numpy as jnp
from jax import lax
from jax.experimental import pallas as pl
from jax.experimental.pallas import tpu as pltpu
from jax.experimental.pallas import tpu_sc as plsc

_N = 8192
_BS, _C, _D, _H, _W = 2, 2, 32, 512, 512
_SZ, _SY, _SX = 7, 15, 15
_SCALE = 10000.0

_NC = 2
_NS = 16
_L = 16

_ROUND_Z = 2
_NROUNDS = _D // _ROUND_Z
_SLAB_WORDS = _C * _ROUND_Z * _H * _W
_TEC_WB = _SLAB_WORDS // _NS
_DHW = _D * _H * _W
_HW = _H * _W
_CHUNK = _N // _NS
_NBUF_ROWS = 60
_ROW_STRIDE = _SX
_BUF_W = _NBUF_ROWS * _ROW_STRIDE + _L
_DUMP = _SLAB_WORDS
_ZERO_W = 16384
_ACC_EXTRA = 7424


def _body(lb, lc, lz, ly, lx, xo, yo, zo, it, pad, cf,
          out,
          acc,
          pad_v, lb_v, lc_v, lz_v, ly_v, lx_v,
          xo_v, yo_v, zo_v, it_v, cf_v,
          zero_v, val_a, idx_a, val_b, idx_b, sem):
    cid = lax.axis_index("c")
    sid = lax.axis_index("s")
    base_e = sid * _CHUNK

    def sload(ref, i):
        return ref[pl.ds(i, _L)][0]

    pltpu.sync_copy(pad, pad_v)
    pltpu.sync_copy(cf, cf_v)
    pltpu.sync_copy(lb.at[pl.ds(base_e, _CHUNK)], lb_v.at[pl.ds(0, _CHUNK)])
    pltpu.sync_copy(lc.at[pl.ds(base_e, _CHUNK)], lc_v.at[pl.ds(0, _CHUNK)])
    pltpu.sync_copy(lz.at[pl.ds(base_e, _CHUNK)], lz_v.at[pl.ds(0, _CHUNK)])
    pltpu.sync_copy(ly.at[pl.ds(base_e, _CHUNK)], ly_v.at[pl.ds(0, _CHUNK)])
    pltpu.sync_copy(lx.at[pl.ds(base_e, _CHUNK)], lx_v.at[pl.ds(0, _CHUNK)])
    pltpu.sync_copy(xo.at[pl.ds(base_e, _CHUNK)], xo_v.at[pl.ds(0, _CHUNK)])
    pltpu.sync_copy(yo.at[pl.ds(base_e, _CHUNK)], yo_v.at[pl.ds(0, _CHUNK)])
    pltpu.sync_copy(zo.at[pl.ds(base_e, _CHUNK)], zo_v.at[pl.ds(0, _CHUNK)])
    pltpu.sync_copy(it.at[pl.ds(base_e, _CHUNK)], it_v.at[pl.ds(0, _CHUNK)])

    ii = lax.iota(jnp.int32, _L)
    zvec = jnp.zeros((_L,), jnp.float32)

    def zb(j, _):
        zero_v[pl.ds(j * _L, _L)] = zvec
        return 0
    lax.fori_loop(0, _ZERO_W // _L, zb, 0)

    dump_idx = _DUMP + ii
    for buf_v, buf_i in ((val_a, idx_a), (val_b, idx_b)):
        buf_v[pl.ds(_NBUF_ROWS * _ROW_STRIDE, _L)] = zvec
        buf_i[pl.ds(_NBUF_ROWS * _ROW_STRIDE, _L)] = dump_idx

    def round_body(r, _):
        z0 = r * _ROUND_Z

        def zr(j, _):
            pltpu.sync_copy(
                zero_v, acc.at[pl.ds(sid * _TEC_WB + j * _ZERO_W, _ZERO_W)])
            return 0
        lax.fori_loop(0, _TEC_WB // _ZERO_W, zr, 0)
        plsc.subcore_barrier()

        def emitter_body(i, carry):
            elz = sload(lz_v, i)
            elb = sload(lb_v, i)
            zlo = jnp.maximum(z0, elz - (_SZ // 2))
            zhi = jnp.minimum(z0 + _ROUND_Z - 1, elz + (_SZ // 2))
            zub = jnp.where(elb == cid, zhi + 1, zlo)

            def z_body(zz, carry):
                rowcnt, fcnt = carry
                elc = sload(lc_v, i)
                ely = sload(ly_v, i)
                elx = sload(lx_v, i)
                dz = sload(zo_v, i) - 0.5
                dy = sload(yo_v, i) - 0.5
                dx = sload(xo_v, i) - 0.5
                fzi = jnp.where(dz < 0.0, -1, 0)
                fyi = jnp.where(dy < 0.0, -1, 0)
                fxi = jnp.where(dx < 0.0, -1, 0)
                wz1 = dz - fzi.astype(jnp.float32)
                wy1 = dy - fyi.astype(jnp.float32)
                wx1 = dx - fxi.astype(jnp.float32)
                wz0 = 1.0 - wz1
                wy0 = 1.0 - wy1
                wx0 = 1.0 - wx1
                cfv = cf_v[...]
                fac = sload(it_v, i) * _SCALE * jnp.where(elc == 0, cfv[0], cfv[1])
                wzf0 = wz0 * fac
                wzf1 = wz1 * fac

                k = zz - elz + (_SZ // 2)
                jz0 = k + 1 + fzi
                jz1 = jz0 + 1
                x0 = 1 + fxi
                x1 = x0 + 1
                sbase_z = (elc * _ROUND_Z + (zz - z0)) * _HW
                xbase = elx - (_SX // 2)
                xv = xbase + ii
                xok = (xv >= 0) & (xv < _W) & (ii < _SX)

                dylo = jnp.maximum(0, (_SY // 2) - ely)
                dyhi = jnp.minimum(_SY - 1, (_H - 1) - ely + (_SY // 2))

                bv = []
                for t in range(_SY + 1):
                    jy = t + 1 + fyi
                    xb0 = (pad_v[jz0, jy, pl.ds(x0, _L)] * wx0 +
                           pad_v[jz0, jy, pl.ds(x1, _L)] * wx1)
                    xb1 = (pad_v[jz1, jy, pl.ds(x0, _L)] * wx0 +
                           pad_v[jz1, jy, pl.ds(x1, _L)] * wx1)
                    bv.append(xb0 * wzf0 + xb1 * wzf1)

                y0 = ely - (_SY // 2)
                idx0 = jnp.where(xok, sbase_z + y0 * _W + xv, dump_idx)
                p = fcnt & 1
                base_pos = rowcnt * _ROW_STRIDE
                rows = []
                for t in range(_SY):
                    row = bv[t] * wy0 + bv[t + 1] * wy1
                    yok = (t >= dylo) & (t <= dyhi)
                    ridx = jnp.where(yok, idx0 + t * _W, dump_idx)
                    rows.append((row, ridx))

                def stage(val_r, idx_r):
                    for t, (row, ridx) in enumerate(rows):
                        val_r[pl.ds(base_pos + t * _ROW_STRIDE, _L)] = row
                        idx_r[pl.ds(base_pos + t * _ROW_STRIDE, _L)] = ridx

                @pl.when(p == 0)
                def _():
                    stage(val_a, idx_a)

                @pl.when(p == 1)
                def _():
                    stage(val_b, idx_b)

                rowcnt = rowcnt + _SY
                full = rowcnt == _NBUF_ROWS

                @pl.when(full & (p == 0))
                def _():
                    pltpu.async_copy(val_a, acc.at[idx_a], sem.at[0],
                                     add=True)

                    @pl.when(fcnt >= 1)
                    def _():
                        pltpu.make_async_copy(val_b, acc.at[idx_b],
                                              sem.at[1]).wait()

                @pl.when(full & (p == 1))
                def _():
                    pltpu.async_copy(val_b, acc.at[idx_b], sem.at[1],
                                     add=True)
                    pltpu.make_async_copy(val_a, acc.at[idx_a],
                                          sem.at[0]).wait()
                return (jnp.where(full, 0, rowcnt),
                        jnp.where(full, fcnt + 1, fcnt))

            return lax.fori_loop(zlo, zub, z_body, carry)

        rowcnt, fcnt = lax.fori_loop(0, _CHUNK, emitter_body, (0, 0))

        pf = fcnt & 1

        @pl.when(pf == 0)
        def _():
            def pad_body(j, _):
                pos = j * _ROW_STRIDE
                val_a[pl.ds(pos, _L)] = zvec
                idx_a[pl.ds(pos, _L)] = dump_idx
                return 0
            lax.fori_loop(rowcnt, _NBUF_ROWS, pad_body, 0)
            pltpu.sync_copy(val_a, acc.at[idx_a], add=True)

            @pl.when(fcnt >= 1)
            def _():
                pltpu.make_async_copy(val_b, acc.at[idx_b], sem.at[1]).wait()

        @pl.when(pf == 1)
        def _():
            def pad_body(j, _):
                pos = j * _ROW_STRIDE
                val_b[pl.ds(pos, _L)] = zvec
                idx_b[pl.ds(pos, _L)] = dump_idx
                return 0
            lax.fori_loop(rowcnt, _NBUF_ROWS, pad_body, 0)
            pltpu.sync_copy(val_b, acc.at[idx_b], add=True)
            pltpu.make_async_copy(val_a, acc.at[idx_a], sem.at[0]).wait()

        plsc.subcore_barrier()

        bcl = sid // 8
        zof = (sid // 4) % 2
        yq = sid % 4
        hbm_off = ((2 * cid + bcl) * _DHW + (z0 + zof) * _HW
                   + yq * (_H // 4) * _W)
        pltpu.sync_copy(acc.at[pl.ds(sid * _TEC_WB, _TEC_WB)],
                        out.at[pl.ds(hbm_off, _TEC_WB)])
        return 0

    lax.fori_loop(0, _NROUNDS, round_body, 0)


@jax.jit
def _sc_place(lb, lc, lz, ly, lx, xo, yo, zo, it, pad, cf):
    mesh = plsc.VectorSubcoreMesh(core_axis_name="c", subcore_axis_name="s",
                                  num_cores=_NC, num_subcores=_NS)
    f = pl.kernel(
        _body,
        out_type=jax.ShapeDtypeStruct((_BS * _C * _D * _H * _W,), jnp.float32),
        mesh=mesh,
        scratch_types=[
            pltpu.VMEM_SHARED((_SLAB_WORDS + _ACC_EXTRA,), jnp.float32),
            pltpu.VMEM((_SZ + 2, _SY + 2, 24), jnp.float32),
            pltpu.VMEM((_CHUNK + _L,), jnp.int32),
            pltpu.VMEM((_CHUNK + _L,), jnp.int32),
            pltpu.VMEM((_CHUNK + _L,), jnp.int32),
            pltpu.VMEM((_CHUNK + _L,), jnp.int32),
            pltpu.VMEM((_CHUNK + _L,), jnp.int32),
            pltpu.VMEM((_CHUNK + _L,), jnp.float32),
            pltpu.VMEM((_CHUNK + _L,), jnp.float32),
            pltpu.VMEM((_CHUNK + _L,), jnp.float32),
            pltpu.VMEM((_CHUNK + _L,), jnp.float32),
            pltpu.VMEM((_L,), jnp.float32),
            pltpu.VMEM((_ZERO_W,), jnp.float32),
            pltpu.VMEM((_BUF_W,), jnp.float32),
            pltpu.VMEM((_BUF_W,), jnp.int32),
            pltpu.VMEM((_BUF_W,), jnp.float32),
            pltpu.VMEM((_BUF_W,), jnp.int32),
            pltpu.SemaphoreType.DMA((2,)),
        ],
    )
    return f(lb, lc, lz, ly, lx, xo, yo, zo, it, pad, cf)


def kernel(loc_b, loc_c, loc_z, loc_y, loc_x, x_os, y_os, z_os, ints,
           psf_volume, channel_facs):
    psfc = jnp.maximum(psf_volume.astype(jnp.float32), 0.0)
    pad = jnp.pad(psfc, ((1, 1), (1, 1), (1, 24 - _SX - 1)))
    cf = jnp.zeros((_L,), jnp.float32).at[:_C].set(
        channel_facs.astype(jnp.float32))
    out = _sc_place(
        loc_b.astype(jnp.int32), loc_c.astype(jnp.int32),
        loc_z.astype(jnp.int32), loc_y.astype(jnp.int32),
        loc_x.astype(jnp.int32),
        x_os.astype(jnp.float32), y_os.astype(jnp.float32),
        z_os.astype(jnp.float32), ints.astype(jnp.float32),
        pad, cf)
    return out.reshape(_BS, _C, _D, _H, _W)
